# baseline (speedup 1.0000x reference)
import functools
import jax
import jax.numpy as jnp
from jax import lax
from jax.experimental import pallas as pl
from jax.experimental.pallas import tpu as pltpu
from jax.experimental.pallas import tpu_sc as plsc

FUSION_THRESHOLD = 0.5
_CH = 64
_NEG = -1e30


def _make_sc_pass(bs, d):
    info = plsc.get_sparse_core_info()
    nc, ns = info.num_cores, info.num_subcores
    nw = nc * ns
    rpw = bs // nw
    nch = rpw // _CH
    nk = d // 16
    mesh = plsc.VectorSubcoreMesh(core_axis_name="c", subcore_axis_name="s")

    @functools.partial(
        pl.kernel,
        mesh=mesh,
        out_type=(
            jax.ShapeDtypeStruct((bs, d), jnp.float32),
            jax.ShapeDtypeStruct((nw, 8, 16), jnp.float32),
        ),
        scratch_types=[
            pltpu.VMEM((_CH, d), jnp.float32),
            pltpu.VMEM((_CH, d), jnp.float32),
            pltpu.VMEM((2, d), jnp.float32),
            pltpu.VMEM((8, d), jnp.float32),
            pltpu.VMEM((8, 16), jnp.float32),
            pltpu.SemaphoreType.DMA,
            pltpu.SemaphoreType.DMA,
            pltpu.SemaphoreType.DMA,
            pltpu.SemaphoreType.DMA,
            pltpu.SemaphoreType.DMA,
        ],
    )
    def k(x_hbm, w_hbm, o_hbm, fl_hbm, buf0, buf1, wbuf, hbuf, fvec,
          si0, si1, so0, so1, saux):
        wid = lax.axis_index("s") * nc + lax.axis_index("c")
        base = wid * rpw
        bufs = (buf0, buf1)
        sin = (si0, si1)
        sout = (so0, so1)
        lane = lax.iota(jnp.int32, 16)

        pltpu.async_copy(w_hbm, wbuf, saux).wait()
        pre_start = pl.multiple_of(jnp.maximum(base - 8, 0), 8)
        pltpu.async_copy(x_hbm.at[pl.ds(pre_start, 8)], hbuf, saux).wait()

        def allsum(v):
            for st in (8, 4, 2, 1):
                v = v + jnp.take(v, lane ^ st, mode="promise_in_bounds")
            return v

        def dots(buf, r):
            acc_a = jnp.zeros((16,), jnp.float32)
            acc_b = jnp.zeros((16,), jnp.float32)
            for kc in range(nk):
                xk = buf[r, pl.ds(kc * 16, 16)]
                acc_a = acc_a + xk * wbuf[0, pl.ds(kc * 16, 16)]
                acc_b = acc_b + xk * wbuf[1, pl.ds(kc * 16, 16)]
            return allsum(acc_a), allsum(acc_b)

        def row_scores(buf, r, carry):
            a_prev, smax = carry
            a_r, b_r = dots(buf, r)
            return a_r, jnp.maximum(smax, a_prev + b_r)

        a_pre, _ = dots(hbuf, 7)
        neg = jnp.full((16,), _NEG, jnp.float32)
        a_prev = jnp.where(wid % 2 == 1, a_pre, neg)

        in_h = [None] * nch
        out_h = [None] * nch
        in_h[0] = pltpu.async_copy(x_hbm.at[pl.ds(base, _CH)], buf0, si0)
        carry = (a_prev, neg)
        for c in range(nch):
            nxt = (c + 1) % 2
            if c + 1 < nch:
                if c - 1 >= 0:
                    out_h[c - 1].wait()
                in_h[c + 1] = pltpu.async_copy(
                    x_hbm.at[pl.ds(base + (c + 1) * _CH, _CH)], bufs[nxt], sin[nxt]
                )
            in_h[c].wait()
            buf = bufs[c % 2]
            carry = lax.fori_loop(
                0, _CH, lambda r, cr: row_scores(buf, r, cr), carry
            )
            out_h[c] = pltpu.async_copy(
                buf, o_hbm.at[pl.ds(base + c * _CH, _CH)], sout[c % 2]
            )
        for i in range(8):
            fvec[i] = carry[1]
        pltpu.async_copy(fvec, fl_hbm.at[wid], saux).wait()
        out_h[nch - 2].wait()
        out_h[nch - 1].wait()

    return k


def _fixup_body(o_in_ref, fl_ref, swt_ref, sb_ref, cw_ref, cb_ref, o_ref,
                xrow, orow, cwv, sem):
    del o_in_ref
    smax = jnp.max(fl_ref[...]) + sb_ref[0, 0]

    @pl.when(smax >= FUSION_THRESHOLD)
    def _():
        cp = pltpu.make_async_copy(cw_ref, cwv, sem)
        cp.start()
        cp.wait()
        s, d = xrow.shape
        nrows = o_ref.shape[0] // s
        for i in range(nrows):
            cx = pltpu.make_async_copy(o_ref.at[pl.ds(i * s, s)], xrow, sem)
            cx.start()
            cx.wait()
            x = xrow[...]
            uv = jax.lax.dot_general(
                x, swt_ref[...], (((1,), (0,)), ((), ())),
                preferred_element_type=jnp.float32,
            )
            u = uv[:, 0:1]
            v = uv[:, 1:2]
            v_next = jnp.concatenate([v[1:], v[-1:]], axis=0)
            score = u + v_next + sb_ref[0, 0]
            row = lax.broadcasted_iota(jnp.int32, (s, 1), 0)
            fuse = (score >= FUSION_THRESHOLD) & (row < s - 1)
            xn = jnp.concatenate([x[1:], x[-1:]], axis=0)
            fused = (
                jnp.dot(x, cwv[0:d, :], preferred_element_type=jnp.float32)
                + jnp.dot(xn, cwv[d:, :], preferred_element_type=jnp.float32)
                + cb_ref[0:1, :]
            )
            orow[...] = jnp.where(fuse, fused, x)
            co = pltpu.make_async_copy(orow, o_ref.at[pl.ds(i * s, s)], sem)
            co.start()
            co.wait()


def kernel(frame_input, score_w, score_b, comb_w, comb_b):
    b, s, d = frame_input.shape
    bs = b * s
    xf = frame_input.reshape(bs, d)
    wts = score_w.reshape(2, d)
    sc_out, flags = _make_sc_pass(bs, d)(xf, wts)
    flags = flags.reshape(-1, 16)

    swt = score_w.reshape(2, d).T
    sb = score_b.reshape(1, 1)
    cb = comb_b.reshape(1, d)
    out = pl.pallas_call(
        _fixup_body,
        grid=(1,),
        in_specs=[
            pl.BlockSpec(memory_space=pl.ANY),
            pl.BlockSpec((flags.shape[0], 16), lambda i: (0, 0)),
            pl.BlockSpec((d, 2), lambda i: (0, 0)),
            pl.BlockSpec(memory_space=pltpu.MemorySpace.SMEM),
            pl.BlockSpec(memory_space=pl.ANY),
            pl.BlockSpec((1, d), lambda i: (0, 0)),
        ],
        out_specs=pl.BlockSpec(memory_space=pl.ANY),
        out_shape=jax.ShapeDtypeStruct((bs, d), jnp.float32),
        scratch_shapes=[
            pltpu.VMEM((s, d), jnp.float32),
            pltpu.VMEM((s, d), jnp.float32),
            pltpu.VMEM((2 * d, d), jnp.float32),
            pltpu.SemaphoreType.DMA,
        ],
        input_output_aliases={0: 0},
    )(sc_out, flags, swt, sb, comb_w, cb)
    return out.reshape(b, s, d)


# speedup vs baseline: 1.1960x; 1.1960x over previous
import functools
import jax
import jax.numpy as jnp
from jax import lax
from jax.experimental import pallas as pl
from jax.experimental.pallas import tpu as pltpu
from jax.experimental.pallas import tpu_sc as plsc

FUSION_THRESHOLD = 0.5
_CH = 64
_NEG = -1e30


def _make_sc_pass(bs, d):
    info = plsc.get_sparse_core_info()
    nc, ns = info.num_cores, info.num_subcores
    nw = nc * ns
    rpw = bs // nw
    nch = rpw // _CH
    nk = d // 16
    mesh = plsc.VectorSubcoreMesh(core_axis_name="c", subcore_axis_name="s")

    @functools.partial(
        pl.kernel,
        mesh=mesh,
        out_type=(
            jax.ShapeDtypeStruct((bs, d), jnp.float32),
            jax.ShapeDtypeStruct((nw, 8, 16), jnp.float32),
        ),
        scratch_types=[
            pltpu.VMEM((_CH, d), jnp.float32),
            pltpu.VMEM((_CH, d), jnp.float32),
            pltpu.VMEM((2, d), jnp.float32),
            pltpu.VMEM((8, d), jnp.float32),
            pltpu.VMEM((8, 16), jnp.float32),
            pltpu.SemaphoreType.DMA,
            pltpu.SemaphoreType.DMA,
            pltpu.SemaphoreType.DMA,
            pltpu.SemaphoreType.DMA,
            pltpu.SemaphoreType.DMA,
        ],
    )
    def k(x_hbm, w_hbm, o_hbm, fl_hbm, buf0, buf1, wbuf, hbuf, fvec,
          si0, si1, so0, so1, saux):
        wid = lax.axis_index("s") * nc + lax.axis_index("c")
        base = wid * rpw
        bufs = (buf0, buf1)
        sin = (si0, si1)
        sout = (so0, so1)
        lane = lax.iota(jnp.int32, 16)

        pltpu.async_copy(w_hbm, wbuf, saux).wait()
        pre_start = pl.multiple_of(jnp.maximum(base - 8, 0), 8)
        pltpu.async_copy(x_hbm.at[pl.ds(pre_start, 8)], hbuf, saux).wait()

        def allsum(v):
            for st in (8, 4, 2, 1):
                v = v + jnp.take(v, lane ^ st, mode="promise_in_bounds")
            return v

        def dots(buf, r):
            acc_a = jnp.zeros((16,), jnp.float32)
            acc_b = jnp.zeros((16,), jnp.float32)
            for kc in range(nk):
                xk = buf[r, pl.ds(kc * 16, 16)]
                acc_a = acc_a + xk * wbuf[0, pl.ds(kc * 16, 16)]
                acc_b = acc_b + xk * wbuf[1, pl.ds(kc * 16, 16)]
            return allsum(acc_a), allsum(acc_b)

        def row_scores(buf, r, carry):
            a_prev, smax = carry
            a_r, b_r = dots(buf, r)
            return a_r, jnp.maximum(smax, a_prev + b_r)

        a_pre, _ = dots(hbuf, 7)
        neg = jnp.full((16,), _NEG, jnp.float32)
        a_prev = jnp.where(wid % 2 == 1, a_pre, neg)

        in_h = [None] * nch
        out_h = [None] * nch
        in_h[0] = pltpu.async_copy(x_hbm.at[pl.ds(base, _CH)], buf0, si0)
        carry = (a_prev, neg)
        for c in range(nch):
            nxt = (c + 1) % 2
            if c + 1 < nch:
                if c - 1 >= 0:
                    out_h[c - 1].wait()
                in_h[c + 1] = pltpu.async_copy(
                    x_hbm.at[pl.ds(base + (c + 1) * _CH, _CH)], bufs[nxt], sin[nxt]
                )
            in_h[c].wait()
            buf = bufs[c % 2]
            out_h[c] = pltpu.async_copy(
                buf, o_hbm.at[pl.ds(base + c * _CH, _CH)], sout[c % 2]
            )
            carry = lax.fori_loop(
                0, _CH, lambda r, cr: row_scores(buf, r, cr), carry
            )
        for i in range(8):
            fvec[i] = carry[1]
        pltpu.async_copy(fvec, fl_hbm.at[wid], saux).wait()
        out_h[nch - 2].wait()
        out_h[nch - 1].wait()

    return k


def _fixup_body(o_in_ref, fl_ref, swt_ref, sb_ref, cw_ref, cb_ref, o_ref,
                xrow, orow, cwv, sem):
    del o_in_ref
    smax = jnp.max(fl_ref[...]) + sb_ref[0, 0]

    @pl.when(smax >= FUSION_THRESHOLD)
    def _():
        cp = pltpu.make_async_copy(cw_ref, cwv, sem)
        cp.start()
        cp.wait()
        s, d = xrow.shape
        nrows = o_ref.shape[0] // s
        for i in range(nrows):
            cx = pltpu.make_async_copy(o_ref.at[pl.ds(i * s, s)], xrow, sem)
            cx.start()
            cx.wait()
            x = xrow[...]
            uv = jax.lax.dot_general(
                x, swt_ref[...], (((1,), (0,)), ((), ())),
                preferred_element_type=jnp.float32,
            )
            u = uv[:, 0:1]
            v = uv[:, 1:2]
            v_next = jnp.concatenate([v[1:], v[-1:]], axis=0)
            score = u + v_next + sb_ref[0, 0]
            row = lax.broadcasted_iota(jnp.int32, (s, 1), 0)
            fuse = (score >= FUSION_THRESHOLD) & (row < s - 1)
            xn = jnp.concatenate([x[1:], x[-1:]], axis=0)
            fused = (
                jnp.dot(x, cwv[0:d, :], preferred_element_type=jnp.float32)
                + jnp.dot(xn, cwv[d:, :], preferred_element_type=jnp.float32)
                + cb_ref[0:1, :]
            )
            orow[...] = jnp.where(fuse, fused, x)
            co = pltpu.make_async_copy(orow, o_ref.at[pl.ds(i * s, s)], sem)
            co.start()
            co.wait()


def kernel(frame_input, score_w, score_b, comb_w, comb_b):
    b, s, d = frame_input.shape
    bs = b * s
    xf = frame_input.reshape(bs, d)
    wts = score_w.reshape(2, d)
    sc_out, flags = _make_sc_pass(bs, d)(xf, wts)
    flags = flags.reshape(-1, 16)

    swt = score_w.reshape(2, d).T
    sb = score_b.reshape(1, 1)
    cb = comb_b.reshape(1, d)
    out = pl.pallas_call(
        _fixup_body,
        grid=(1,),
        in_specs=[
            pl.BlockSpec(memory_space=pl.ANY),
            pl.BlockSpec((flags.shape[0], 16), lambda i: (0, 0)),
            pl.BlockSpec((d, 2), lambda i: (0, 0)),
            pl.BlockSpec(memory_space=pltpu.MemorySpace.SMEM),
            pl.BlockSpec(memory_space=pl.ANY),
            pl.BlockSpec((1, d), lambda i: (0, 0)),
        ],
        out_specs=pl.BlockSpec(memory_space=pl.ANY),
        out_shape=jax.ShapeDtypeStruct((bs, d), jnp.float32),
        scratch_shapes=[
            pltpu.VMEM((s, d), jnp.float32),
            pltpu.VMEM((s, d), jnp.float32),
            pltpu.VMEM((2 * d, d), jnp.float32),
            pltpu.SemaphoreType.DMA,
        ],
        input_output_aliases={0: 0},
    )(sc_out, flags, swt, sb, comb_w, cb)
    return out.reshape(b, s, d)


# speedup vs baseline: 2.3080x; 1.9298x over previous
import jax
import jax.numpy as jnp
from jax.experimental import pallas as pl
from jax.experimental.pallas import tpu as pltpu

FUSION_THRESHOLD = 0.5
_BB = 2


def _fusion_body(x_ref, sw_ref, sb_ref, cw_ref, cb_ref, o_ref):
    x = x_ref[...]
    bb, s, d = x.shape
    uv = jax.lax.dot_general(
        x, sw_ref[...], (((2,), (0,)), ((), ())),
        preferred_element_type=jnp.float32,
    )
    u = uv[:, :, 0:1]
    v = uv[:, :, 1:2]
    v_next = jnp.concatenate([v[:, 1:], v[:, -1:]], axis=1)
    score = u + v_next + sb_ref[0, 0]
    row = jax.lax.broadcasted_iota(jnp.int32, (bb, s, 1), 1)
    fuse = (score >= FUSION_THRESHOLD) & (row < s - 1)
    o_ref[...] = x

    @pl.when(jnp.any(fuse))
    def _():
        xn = jnp.concatenate([x[:, 1:], x[:, -1:]], axis=1)
        fused = (
            jax.lax.dot_general(
                x, cw_ref[0:d, :], (((2,), (0,)), ((), ())),
                preferred_element_type=jnp.float32,
            )
            + jax.lax.dot_general(
                xn, cw_ref[d:, :], (((2,), (0,)), ((), ())),
                preferred_element_type=jnp.float32,
            )
            + cb_ref[0:1, :]
        )
        o_ref[...] = jnp.where(fuse, fused, x)


def kernel(frame_input, score_w, score_b, comb_w, comb_b):
    b, s, d = frame_input.shape
    bb = _BB if b % _BB == 0 else 1
    sw = score_w.reshape(2, d).T
    sb = score_b.reshape(1, 1)
    cb = comb_b.reshape(1, d)
    return pl.pallas_call(
        _fusion_body,
        grid=(b // bb,),
        in_specs=[
            pl.BlockSpec((bb, s, d), lambda i: (i, 0, 0)),
            pl.BlockSpec((d, 2), lambda i: (0, 0)),
            pl.BlockSpec(memory_space=pltpu.SMEM),
            pl.BlockSpec((2 * d, d), lambda i: (0, 0)),
            pl.BlockSpec((1, d), lambda i: (0, 0)),
        ],
        out_specs=pl.BlockSpec((bb, s, d), lambda i: (i, 0, 0)),
        out_shape=jax.ShapeDtypeStruct((b, s, d), frame_input.dtype),
        compiler_params=pltpu.CompilerParams(dimension_semantics=("parallel",)),
    )(frame_input, sw, sb, comb_w, cb)
